# baseline (speedup 1.0000x reference)
import functools

import jax
import jax.numpy as jnp
from jax import lax
from jax.experimental import pallas as pl
from jax.experimental.pallas import tpu as pltpu
from jax.experimental.pallas import tpu_sc as plsc

_B, _C, _Y, _X = 8, 64, 224, 224
_FC = 4
_NBC = _B * _FC
_OY, _OX = 32, 32
_TY = _Y // 8
_FRAG_PER_IMG = _TY * 16
_NFRAG = _B * _C * _FRAG_PER_IMG


def _stats_body(fm_ref, gxr_ref, gyc_ref, g32r_ref,
                mx_ref, my_ref, sc_ref, rid_ref, xm_ref, w_ref):
    bc = pl.program_id(0)
    f = fm_ref[0]
    m = jnp.max(f)
    e = jnp.exp(f - m)
    col = jnp.sum(e, axis=0, keepdims=True)
    row = jnp.sum(e, axis=1, keepdims=True)
    s_tot = jnp.sum(col)
    gx = gxr_ref[...]
    gy = gyc_ref[...]
    mean_x = jnp.sum(col * gx) / s_tot
    mean_y = jnp.sum(row * gy) / s_tot
    scale = (jnp.sum(col * jnp.abs(gx - mean_x))
             + jnp.sum(row * jnp.abs(gy - mean_y))) / s_tot

    g32r = g32r_ref[...]
    x_raw = ((g32r * scale + mean_x) + 1.0) * (_X / 2.0)
    xu = jnp.clip(jnp.ceil(x_raw), 0.0, _X - 1.0)
    xl = jnp.clip(jnp.floor(x_raw), 0.0, _X - 1.0)
    wxu = x_raw - xl
    wxl = xu - x_raw
    y_raw = ((g32r * scale + mean_y) + 1.0) * (_Y / 2.0)
    yu = jnp.clip(jnp.ceil(y_raw), 0.0, _Y - 1.0)
    yl = jnp.clip(jnp.floor(y_raw), 0.0, _Y - 1.0)
    wyu = y_raw - yl
    wyl = yu - y_raw

    mx_ref[...] = jnp.full((1, 1, 128), mean_x, jnp.float32)
    my_ref[...] = jnp.full((1, 1, 128), mean_y, jnp.float32)
    sc_ref[...] = jnp.full((1, 1, 128), scale, jnp.float32)

    b = bc // _FC
    yconc = jnp.concatenate([yu, yl], axis=1).astype(jnp.int32)
    chv = lax.broadcasted_iota(jnp.int32, (_C, 1), 0)
    rid = (b * _C + chv) * _Y + yconc
    rid_ref[...] = rid

    xui = xu.astype(jnp.int32)
    xli = xl.astype(jnp.int32)
    half = _X // 2
    xm = jnp.concatenate(
        [jnp.where(xui < half, xui, xui - half),
         jnp.where(xui < half, 16, 0),
         jnp.where(xli < half, xli, xli - half),
         jnp.where(xli < half, 16, 0)], axis=1)
    xm_ref[...] = xm.reshape(1, 1, 128)

    w = jnp.concatenate([wxu, wxl, wyu, wyl], axis=1)
    w_ref[...] = w.reshape(1, 1, 128)


def _tc_stats(fm32, gxr, gyc, g32r):
    n = _NBC
    f32 = jnp.float32
    i32 = jnp.int32
    outs = [
        jax.ShapeDtypeStruct((n, 1, 128), f32),
        jax.ShapeDtypeStruct((n, 1, 128), f32),
        jax.ShapeDtypeStruct((n, 1, 128), f32),
        jax.ShapeDtypeStruct((n * _C, 64), i32),
        jax.ShapeDtypeStruct((n, 1, 128), i32),
        jax.ShapeDtypeStruct((n, 1, 128), f32),
    ]
    full = lambda shp: pl.BlockSpec(shp, lambda i: (0, 0))
    blk3 = lambda shp: pl.BlockSpec(shp, lambda i: (i, 0, 0))
    return pl.pallas_call(
        _stats_body,
        grid=(n,),
        in_specs=[
            pl.BlockSpec((1, _Y, _X), lambda i: (i, 0, 0)),
            full((1, _X)), full((_Y, 1)), full((1, _OX)),
        ],
        out_specs=[
            blk3((1, 1, 128)), blk3((1, 1, 128)), blk3((1, 1, 128)),
            pl.BlockSpec((_C, 64), lambda i: (i, 0)),
            blk3((1, 1, 128)), blk3((1, 1, 128)),
        ],
        out_shape=outs,
    )(fm32, gxr, gyc, g32r)


_RB = 16


def _retile_body(img_ref, out_ref):
    half = _X // 2
    for k in range(_RB):
        img = img_ref[k]
        u = lax.bitcast_convert_type(
            img.astype(jnp.bfloat16), jnp.uint16).astype(jnp.int32)
        pk = jnp.bitwise_or(u[:, :half],
                            lax.shift_left(u[:, half:], 16))
        row = jnp.concatenate(
            [pk, jnp.zeros((_Y, 16), jnp.int32)], axis=1)
        out_ref[pl.ds(k * _Y, _Y), :] = row


def _tc_retile(img3):
    return pl.pallas_call(
        _retile_body,
        grid=(_B * _C // _RB,),
        in_specs=[pl.BlockSpec((_RB, _Y, _X), lambda i: (i, 0, 0))],
        out_specs=pl.BlockSpec((_RB * _Y, 128), lambda i: (i, 0)),
        out_shape=jax.ShapeDtypeStruct((_B * _C * _Y, 128), jnp.int32),
    )(img3)


def _sc_bilinear(frags, rid, xm, w):
    mesh = plsc.VectorSubcoreMesh(core_axis_name="c", subcore_axis_name="s")

    @functools.partial(
        pl.kernel,
        mesh=mesh,
        compiler_params=pltpu.CompilerParams(use_tc_tiling_on_sc=True,
                                             needs_layout_passes=False),
        out_type=jax.ShapeDtypeStruct((_B * _C * _FC * 8, 128), jnp.float32),
        scratch_types=[
            pltpu.VMEM((_C, 64), jnp.int32),
            [pltpu.VMEM((64, 128), jnp.int32)] * 4,
            pltpu.VMEM((1, 128), jnp.int32),
            pltpu.VMEM((1, 128), jnp.float32),
            [pltpu.VMEM((8, 128), jnp.float32)] * 4,
            [pltpu.SemaphoreType.DMA] * 4,
            [pltpu.SemaphoreType.DMA] * 4,
        ],
    )
    def body(frag_hbm, rid_hbm, xm_hbm, w_hbm, out_hbm, rid_v, rows_bufs,
             xm_v, w_v, orow_bufs, gsems, osems):
        wid = lax.axis_index("s") * 2 + lax.axis_index("c")
        b = wid // _FC
        c = wid % _FC
        obase = (b * _C) * _FC + c
        pltpu.sync_copy(rid_hbm.at[pl.ds(wid * _C, _C)], rid_v)
        pltpu.sync_copy(xm_hbm.at[wid], xm_v)
        pltpu.sync_copy(w_hbm.at[wid], w_v)

        hmask = jnp.int32(-65536)

        def unpack(v32, sh):
            return plsc.bitcast(
                jnp.bitwise_and(lax.shift_left(v32, sh), hmask), jnp.float32)

        def combine(rows_v, orow_v):
            for h in range(2):
                xcu = xm_v[0, pl.ds(h * 16, 16)]
                shu = xm_v[0, pl.ds(32 + h * 16, 16)]
                xcl = xm_v[0, pl.ds(64 + h * 16, 16)]
                shl = xm_v[0, pl.ds(96 + h * 16, 16)]
                wxu = w_v[0, pl.ds(h * 16, 16)]
                wxl = w_v[0, pl.ds(32 + h * 16, 16)]
                for g in range(_OY // 16):
                    wyu_blk = w_v[0, pl.ds(64 + 16 * g, 16)]
                    wyl_blk = w_v[0, pl.ds(96 + 16 * g, 16)]
                    for ii in range(16):
                        i = g * 16 + ii
                        ru = jnp.full((16,), i, jnp.int32)
                        rl = jnp.full((16,), _OY + i, jnp.int32)
                        puu = unpack(plsc.load_gather(rows_v, [ru, xcu]), shu)
                        pul = unpack(plsc.load_gather(rows_v, [ru, xcl]), shl)
                        plu = unpack(plsc.load_gather(rows_v, [rl, xcu]), shu)
                        pll = unpack(plsc.load_gather(rows_v, [rl, xcl]), shl)
                        iu = wxu * puu + wxl * pul
                        il = wxu * plu + wxl * pll
                        o = wyu_blk[ii] * iu + wyl_blk[ii] * il
                        p = i * _OX + h * 16
                        orow_v[p >> 7, pl.ds(p & 127, 16)] = o

        def wait_gather(rows_v, gsem):
            pltpu.make_async_copy(frag_hbm.at[rid_v.at[0]], rows_v,
                                  gsem).wait()

        def drain_out(orow_v, osem):
            pltpu.make_async_copy(out_hbm.at[pl.ds(0, 8)], orow_v,
                                  osem).wait()

        nbuf = 4
        for j in range(nbuf):
            pltpu.async_copy(frag_hbm.at[rid_v.at[j]], rows_bufs[j], gsems[j])

        def step(k, carry):
            for j in range(nbuf):
                ch = nbuf * k + j
                wait_gather(rows_bufs[j], gsems[j])

                @pl.when(k > 0)
                def _():
                    drain_out(orow_bufs[j], osems[j])
                combine(rows_bufs[j], orow_bufs[j])
                pltpu.async_copy(orow_bufs[j],
                                 out_hbm.at[pl.ds((obase + ch * _FC) * 8, 8)],
                                 osems[j])

                @pl.when(k < _C // nbuf - 1)
                def _():
                    pltpu.async_copy(frag_hbm.at[rid_v.at[ch + nbuf]],
                                     rows_bufs[j], gsems[j])
            return carry

        lax.fori_loop(0, _C // nbuf, step, 0)
        for j in range(nbuf):
            drain_out(orow_bufs[j], osems[j])

    return body(frags, rid, xm, w)


def kernel(images, feature_map):
    f32 = jnp.float32
    fm32 = feature_map.reshape(_NBC, _Y, _X)
    gxr = jnp.linspace(-1.0, 1.0, _X, dtype=f32).reshape(1, _X)
    gyc = jnp.linspace(-1.0, 1.0, _Y, dtype=f32).reshape(_Y, 1)
    g32r = jnp.linspace(-1.0, 1.0, _OX, dtype=f32).reshape(1, _OX)

    mx, my, sc, rid, xm, w = _tc_stats(fm32, gxr, gyc, g32r)
    frags = _tc_retile(images.reshape(_B * _C, _Y, _X))

    out2d = _sc_bilinear(frags, rid, xm, w)

    out = out2d.reshape(_B, _C, _FC, _OY, _OX)
    mean_x = mx[:, 0, 0].reshape(_B, _FC)
    mean_y = my[:, 0, 0].reshape(_B, _FC)
    scale = sc[:, 0, 0].reshape(_B, _FC)
    return (out, mean_x, mean_y, scale)


# speedup vs baseline: 1.0994x; 1.0994x over previous
import functools

import jax
import jax.numpy as jnp
from jax import lax
from jax.experimental import pallas as pl
from jax.experimental.pallas import tpu as pltpu
from jax.experimental.pallas import tpu_sc as plsc

_B, _C, _Y, _X = 8, 64, 224, 224
_FC = 4
_NBC = _B * _FC
_OY, _OX = 32, 32
_TY = _Y // 8
_FRAG_PER_IMG = _TY * 16
_NFRAG = _B * _C * _FRAG_PER_IMG


def _stats_body(fm_ref, gxr_ref, gyc_ref, g32r_ref,
                mx_ref, my_ref, sc_ref, rid_ref, xm_ref, w_ref):
    bc = pl.program_id(0)
    f = fm_ref[0]
    m = jnp.max(f)
    e = jnp.exp(f - m)
    col = jnp.sum(e, axis=0, keepdims=True)
    row = jnp.sum(e, axis=1, keepdims=True)
    s_tot = jnp.sum(col)
    gx = gxr_ref[...]
    gy = gyc_ref[...]
    mean_x = jnp.sum(col * gx) / s_tot
    mean_y = jnp.sum(row * gy) / s_tot
    scale = (jnp.sum(col * jnp.abs(gx - mean_x))
             + jnp.sum(row * jnp.abs(gy - mean_y))) / s_tot

    g32r = g32r_ref[...]
    x_raw = ((g32r * scale + mean_x) + 1.0) * (_X / 2.0)
    xu = jnp.clip(jnp.ceil(x_raw), 0.0, _X - 1.0)
    xl = jnp.clip(jnp.floor(x_raw), 0.0, _X - 1.0)
    wxu = x_raw - xl
    wxl = xu - x_raw
    y_raw = ((g32r * scale + mean_y) + 1.0) * (_Y / 2.0)
    yu = jnp.clip(jnp.ceil(y_raw), 0.0, _Y - 1.0)
    yl = jnp.clip(jnp.floor(y_raw), 0.0, _Y - 1.0)
    wyu = y_raw - yl
    wyl = yu - y_raw

    mx_ref[...] = jnp.full((1, 1, 128), mean_x, jnp.float32)
    my_ref[...] = jnp.full((1, 1, 128), mean_y, jnp.float32)
    sc_ref[...] = jnp.full((1, 1, 128), scale, jnp.float32)

    b = bc // _FC
    yconc = jnp.concatenate([yu, yl], axis=1).astype(jnp.int32)
    chv = lax.broadcasted_iota(jnp.int32, (_C, 1), 0)
    rid = (b * _C + chv) * _Y + yconc
    rid_ref[...] = rid

    xui = xu.astype(jnp.int32)
    xli = xl.astype(jnp.int32)
    half = _X // 2
    xm = jnp.concatenate(
        [jnp.where(xui < half, xui, xui - half),
         jnp.where(xui < half, 16, 0),
         jnp.where(xli < half, xli, xli - half),
         jnp.where(xli < half, 16, 0)], axis=1)
    xm_ref[...] = xm.reshape(1, 1, 128)

    w = jnp.concatenate([wxu, wxl, wyu, wyl], axis=1)
    w_ref[...] = w.reshape(1, 1, 128)


def _tc_stats(fm32, gxr, gyc, g32r):
    n = _NBC
    f32 = jnp.float32
    i32 = jnp.int32
    outs = [
        jax.ShapeDtypeStruct((n, 1, 128), f32),
        jax.ShapeDtypeStruct((n, 1, 128), f32),
        jax.ShapeDtypeStruct((n, 1, 128), f32),
        jax.ShapeDtypeStruct((n * _C, 64), i32),
        jax.ShapeDtypeStruct((n, 1, 128), i32),
        jax.ShapeDtypeStruct((n, 1, 128), f32),
    ]
    full = lambda shp: pl.BlockSpec(shp, lambda i: (0, 0))
    blk3 = lambda shp: pl.BlockSpec(shp, lambda i: (i, 0, 0))
    return pl.pallas_call(
        _stats_body,
        grid=(n,),
        in_specs=[
            pl.BlockSpec((1, _Y, _X), lambda i: (i, 0, 0)),
            full((1, _X)), full((_Y, 1)), full((1, _OX)),
        ],
        out_specs=[
            blk3((1, 1, 128)), blk3((1, 1, 128)), blk3((1, 1, 128)),
            pl.BlockSpec((_C, 64), lambda i: (i, 0)),
            blk3((1, 1, 128)), blk3((1, 1, 128)),
        ],
        out_shape=outs,
    )(fm32, gxr, gyc, g32r)


_RB = 16


def _retile_body(img_ref, out_ref):
    half = _X // 2
    for k in range(_RB):
        img = img_ref[k]
        u = lax.bitcast_convert_type(
            img.astype(jnp.bfloat16), jnp.uint16).astype(jnp.int32)
        pk = jnp.bitwise_or(u[:, :half],
                            lax.shift_left(u[:, half:], 16))
        row = jnp.concatenate(
            [pk, jnp.zeros((_Y, 16), jnp.int32)], axis=1)
        out_ref[pl.ds(k * _Y, _Y), :] = row


def _tc_retile(img3):
    return pl.pallas_call(
        _retile_body,
        grid=(_B * _C // _RB,),
        in_specs=[pl.BlockSpec((_RB, _Y, _X), lambda i: (i, 0, 0))],
        out_specs=pl.BlockSpec((_RB * _Y, 128), lambda i: (i, 0)),
        out_shape=jax.ShapeDtypeStruct((_B * _C * _Y, 128), jnp.int32),
    )(img3)


def _sc_bilinear(frags, rid, xm, w):
    mesh = plsc.VectorSubcoreMesh(core_axis_name="c", subcore_axis_name="s")

    @functools.partial(
        pl.kernel,
        mesh=mesh,
        compiler_params=pltpu.CompilerParams(use_tc_tiling_on_sc=True,
                                             needs_layout_passes=False),
        out_type=jax.ShapeDtypeStruct((_B, _C, _FC, _OY, _OX), jnp.float32),
        scratch_types=[
            pltpu.VMEM((_C, 64), jnp.int32),
            [pltpu.VMEM((64, 128), jnp.int32)] * 2,
            pltpu.VMEM((1, 128), jnp.int32),
            pltpu.VMEM((1, 128), jnp.float32),
            [pltpu.VMEM((_OY, _OX), jnp.float32)] * 2,
            [pltpu.SemaphoreType.DMA] * 2,
            [pltpu.SemaphoreType.DMA] * 2,
        ],
    )
    def body(frag_hbm, rid_hbm, xm_hbm, w_hbm, out_hbm, rid_v, rows_bufs,
             xm_v, w_v, orow_bufs, gsems, osems):
        wid = lax.axis_index("s") * 2 + lax.axis_index("c")
        b = wid // _FC
        c = wid % _FC
        obase = (b * _C) * _FC + c
        pltpu.sync_copy(rid_hbm.at[pl.ds(wid * _C, _C)], rid_v)
        pltpu.sync_copy(xm_hbm.at[wid], xm_v)
        pltpu.sync_copy(w_hbm.at[wid], w_v)

        hmask = jnp.int32(-65536)

        def unpack(v32, sh):
            return plsc.bitcast(
                jnp.bitwise_and(lax.shift_left(v32, sh), hmask), jnp.float32)

        def combine(rows_v, orow_v):
            for h in range(2):
                xcu = xm_v[0, pl.ds(h * 16, 16)]
                shu = xm_v[0, pl.ds(32 + h * 16, 16)]
                xcl = xm_v[0, pl.ds(64 + h * 16, 16)]
                shl = xm_v[0, pl.ds(96 + h * 16, 16)]
                wxu = w_v[0, pl.ds(h * 16, 16)]
                wxl = w_v[0, pl.ds(32 + h * 16, 16)]
                for g in range(_OY // 16):
                    wyu_blk = w_v[0, pl.ds(64 + 16 * g, 16)]
                    wyl_blk = w_v[0, pl.ds(96 + 16 * g, 16)]
                    for ii in range(16):
                        i = g * 16 + ii
                        ru = jnp.full((16,), i, jnp.int32)
                        rl = jnp.full((16,), _OY + i, jnp.int32)
                        puu = unpack(plsc.load_gather(rows_v, [ru, xcu]), shu)
                        pul = unpack(plsc.load_gather(rows_v, [ru, xcl]), shl)
                        plu = unpack(plsc.load_gather(rows_v, [rl, xcu]), shu)
                        pll = unpack(plsc.load_gather(rows_v, [rl, xcl]), shl)
                        iu = wxu * puu + wxl * pul
                        il = wxu * plu + wxl * pll
                        o = wyu_blk[ii] * iu + wyl_blk[ii] * il
                        orow_v[i, pl.ds(h * 16, 16)] = o

        def wait_gather(rows_v, gsem):
            pltpu.make_async_copy(frag_hbm.at[rid_v.at[0]], rows_v,
                                  gsem).wait()

        def drain_out(orow_v, osem):
            pltpu.make_async_copy(out_hbm.at[0, 0, 0], orow_v, osem).wait()

        nbuf = 2
        for j in range(nbuf):
            pltpu.async_copy(frag_hbm.at[rid_v.at[j]], rows_bufs[j], gsems[j])

        def step(k, carry):
            for j in range(nbuf):
                ch = nbuf * k + j
                wait_gather(rows_bufs[j], gsems[j])

                @pl.when(k > 0)
                def _():
                    drain_out(orow_bufs[j], osems[j])
                combine(rows_bufs[j], orow_bufs[j])
                pltpu.async_copy(orow_bufs[j], out_hbm.at[b, ch, c],
                                 osems[j])

                @pl.when(k < _C // nbuf - 1)
                def _():
                    pltpu.async_copy(frag_hbm.at[rid_v.at[ch + nbuf]],
                                     rows_bufs[j], gsems[j])
            return carry

        lax.fori_loop(0, _C // nbuf, step, 0)
        for j in range(nbuf):
            drain_out(orow_bufs[j], osems[j])

    return body(frags, rid, xm, w)


def kernel(images, feature_map):
    f32 = jnp.float32
    fm32 = feature_map.reshape(_NBC, _Y, _X)
    gxr = jnp.linspace(-1.0, 1.0, _X, dtype=f32).reshape(1, _X)
    gyc = jnp.linspace(-1.0, 1.0, _Y, dtype=f32).reshape(_Y, 1)
    g32r = jnp.linspace(-1.0, 1.0, _OX, dtype=f32).reshape(1, _OX)

    mx, my, sc, rid, xm, w = _tc_stats(fm32, gxr, gyc, g32r)
    frags = _tc_retile(images.reshape(_B * _C, _Y, _X))

    out = _sc_bilinear(frags, rid, xm, w)
    mean_x = mx[:, 0, 0].reshape(_B, _FC)
    mean_y = my[:, 0, 0].reshape(_B, _FC)
    scale = sc[:, 0, 0].reshape(_B, _FC)
    return (out, mean_x, mean_y, scale)


# speedup vs baseline: 1.1163x; 1.0154x over previous
import functools

import jax
import jax.numpy as jnp
from jax import lax
from jax.experimental import pallas as pl
from jax.experimental.pallas import tpu as pltpu
from jax.experimental.pallas import tpu_sc as plsc

_B, _C, _Y, _X = 8, 64, 224, 224
_FC = 4
_NBC = _B * _FC
_OY, _OX = 32, 32
_TY = _Y // 8
_FRAG_PER_IMG = _TY * 16
_NFRAG = _B * _C * _FRAG_PER_IMG


def _stats_body(fm_ref, gxr_ref, gyc_ref, g32r_ref,
                mx_ref, my_ref, sc_ref, rid_ref, xm_ref, w_ref):
    bc = pl.program_id(0)
    f = fm_ref[0]
    m = jnp.max(f)
    e = jnp.exp(f - m)
    col = jnp.sum(e, axis=0, keepdims=True)
    row = jnp.sum(e, axis=1, keepdims=True)
    s_tot = jnp.sum(col)
    gx = gxr_ref[...]
    gy = gyc_ref[...]
    mean_x = jnp.sum(col * gx) / s_tot
    mean_y = jnp.sum(row * gy) / s_tot
    scale = (jnp.sum(col * jnp.abs(gx - mean_x))
             + jnp.sum(row * jnp.abs(gy - mean_y))) / s_tot

    g32r = g32r_ref[...]
    x_raw = ((g32r * scale + mean_x) + 1.0) * (_X / 2.0)
    xu = jnp.clip(jnp.ceil(x_raw), 0.0, _X - 1.0)
    xl = jnp.clip(jnp.floor(x_raw), 0.0, _X - 1.0)
    wxu = x_raw - xl
    wxl = xu - x_raw
    y_raw = ((g32r * scale + mean_y) + 1.0) * (_Y / 2.0)
    yu = jnp.clip(jnp.ceil(y_raw), 0.0, _Y - 1.0)
    yl = jnp.clip(jnp.floor(y_raw), 0.0, _Y - 1.0)
    wyu = y_raw - yl
    wyl = yu - y_raw

    mx_ref[...] = jnp.full((1, 1, 128), mean_x, jnp.float32)
    my_ref[...] = jnp.full((1, 1, 128), mean_y, jnp.float32)
    sc_ref[...] = jnp.full((1, 1, 128), scale, jnp.float32)

    b = bc // _FC
    yconc = jnp.concatenate([yu, yl], axis=1).astype(jnp.int32)
    chv = lax.broadcasted_iota(jnp.int32, (_C, 1), 0)
    rid = (b * _C + chv) * _Y + yconc
    rid_ref[...] = rid

    xui = xu.astype(jnp.int32)
    xli = xl.astype(jnp.int32)
    half = _X // 2
    xm = jnp.concatenate(
        [jnp.where(xui < half, xui, xui - half),
         jnp.where(xui < half, 16, 0),
         jnp.where(xli < half, xli, xli - half),
         jnp.where(xli < half, 16, 0)], axis=1)
    xm_ref[...] = xm.reshape(1, 1, 128)

    w = jnp.concatenate([wxu, wxl, wyu, wyl], axis=1)
    w_ref[...] = w.reshape(1, 1, 128)


def _tc_stats(fm32, gxr, gyc, g32r):
    n = _NBC
    f32 = jnp.float32
    i32 = jnp.int32
    outs = [
        jax.ShapeDtypeStruct((n, 1, 128), f32),
        jax.ShapeDtypeStruct((n, 1, 128), f32),
        jax.ShapeDtypeStruct((n, 1, 128), f32),
        jax.ShapeDtypeStruct((n * _C, 64), i32),
        jax.ShapeDtypeStruct((n, 1, 128), i32),
        jax.ShapeDtypeStruct((n, 1, 128), f32),
    ]
    full = lambda shp: pl.BlockSpec(shp, lambda i: (0, 0))
    blk3 = lambda shp: pl.BlockSpec(shp, lambda i: (i, 0, 0))
    return pl.pallas_call(
        _stats_body,
        grid=(n,),
        in_specs=[
            pl.BlockSpec((1, _Y, _X), lambda i: (i, 0, 0)),
            full((1, _X)), full((_Y, 1)), full((1, _OX)),
        ],
        out_specs=[
            blk3((1, 1, 128)), blk3((1, 1, 128)), blk3((1, 1, 128)),
            pl.BlockSpec((_C, 64), lambda i: (i, 0)),
            blk3((1, 1, 128)), blk3((1, 1, 128)),
        ],
        out_shape=outs,
    )(fm32, gxr, gyc, g32r)


_RB = 32


def _retile_body(img_ref, out_ref):
    half = _X // 2
    for k in range(_RB):
        img = img_ref[k]
        u = lax.bitcast_convert_type(
            img.astype(jnp.bfloat16), jnp.uint16).astype(jnp.int32)
        pk = jnp.bitwise_or(u[:, :half],
                            lax.shift_left(u[:, half:], 16))
        row = jnp.concatenate(
            [pk, jnp.zeros((_Y, 16), jnp.int32)], axis=1)
        out_ref[pl.ds(k * _Y, _Y), :] = row


def _tc_retile(img3):
    return pl.pallas_call(
        _retile_body,
        grid=(_B * _C // _RB,),
        in_specs=[pl.BlockSpec((_RB, _Y, _X), lambda i: (i, 0, 0))],
        out_specs=pl.BlockSpec((_RB * _Y, 128), lambda i: (i, 0)),
        out_shape=jax.ShapeDtypeStruct((_B * _C * _Y, 128), jnp.int32),
    )(img3)


def _sc_bilinear(frags, rid, xm, w):
    mesh = plsc.VectorSubcoreMesh(core_axis_name="c", subcore_axis_name="s")

    @functools.partial(
        pl.kernel,
        mesh=mesh,
        compiler_params=pltpu.CompilerParams(use_tc_tiling_on_sc=True,
                                             needs_layout_passes=False),
        out_type=jax.ShapeDtypeStruct((_B, _C, _FC, _OY, _OX), jnp.float32),
        scratch_types=[
            pltpu.VMEM((_C, 64), jnp.int32),
            [pltpu.VMEM((64, 128), jnp.int32)] * 2,
            pltpu.VMEM((1, 128), jnp.int32),
            pltpu.VMEM((1, 128), jnp.float32),
            [pltpu.VMEM((_OY, _OX), jnp.float32)] * 2,
            [pltpu.SemaphoreType.DMA] * 2,
            [pltpu.SemaphoreType.DMA] * 2,
        ],
    )
    def body(frag_hbm, rid_hbm, xm_hbm, w_hbm, out_hbm, rid_v, rows_bufs,
             xm_v, w_v, orow_bufs, gsems, osems):
        wid = lax.axis_index("s") * 2 + lax.axis_index("c")
        b = wid // _FC
        c = wid % _FC
        obase = (b * _C) * _FC + c
        pltpu.sync_copy(rid_hbm.at[pl.ds(wid * _C, _C)], rid_v)
        pltpu.sync_copy(xm_hbm.at[wid], xm_v)
        pltpu.sync_copy(w_hbm.at[wid], w_v)

        hmask = jnp.int32(-65536)

        def unpack(v32, sh):
            return plsc.bitcast(
                jnp.bitwise_and(lax.shift_left(v32, sh), hmask), jnp.float32)

        def combine(rows_v, orow_v):
            for h in range(2):
                xcu = xm_v[0, pl.ds(h * 16, 16)]
                shu = xm_v[0, pl.ds(32 + h * 16, 16)]
                xcl = xm_v[0, pl.ds(64 + h * 16, 16)]
                shl = xm_v[0, pl.ds(96 + h * 16, 16)]
                wxu = w_v[0, pl.ds(h * 16, 16)]
                wxl = w_v[0, pl.ds(32 + h * 16, 16)]
                for g in range(_OY // 16):
                    wyu_blk = w_v[0, pl.ds(64 + 16 * g, 16)]
                    wyl_blk = w_v[0, pl.ds(96 + 16 * g, 16)]
                    for ii in range(16):
                        i = g * 16 + ii
                        ru = jnp.full((16,), i, jnp.int32)
                        rl = jnp.full((16,), _OY + i, jnp.int32)
                        puu = unpack(plsc.load_gather(rows_v, [ru, xcu]), shu)
                        pul = unpack(plsc.load_gather(rows_v, [ru, xcl]), shl)
                        plu = unpack(plsc.load_gather(rows_v, [rl, xcu]), shu)
                        pll = unpack(plsc.load_gather(rows_v, [rl, xcl]), shl)
                        iu = wxu * puu + wxl * pul
                        il = wxu * plu + wxl * pll
                        o = wyu_blk[ii] * iu + wyl_blk[ii] * il
                        orow_v[i, pl.ds(h * 16, 16)] = o

        def wait_gather(rows_v, gsem):
            pltpu.make_async_copy(frag_hbm.at[rid_v.at[0]], rows_v,
                                  gsem).wait()

        def drain_out(orow_v, osem):
            pltpu.make_async_copy(out_hbm.at[0, 0, 0], orow_v, osem).wait()

        nbuf = 2
        for j in range(nbuf):
            pltpu.async_copy(frag_hbm.at[rid_v.at[j]], rows_bufs[j], gsems[j])

        def step(k, carry):
            for j in range(nbuf):
                ch = nbuf * k + j
                wait_gather(rows_bufs[j], gsems[j])

                @pl.when(k > 0)
                def _():
                    drain_out(orow_bufs[j], osems[j])
                combine(rows_bufs[j], orow_bufs[j])
                pltpu.async_copy(orow_bufs[j], out_hbm.at[b, ch, c],
                                 osems[j])

                @pl.when(k < _C // nbuf - 1)
                def _():
                    pltpu.async_copy(frag_hbm.at[rid_v.at[ch + nbuf]],
                                     rows_bufs[j], gsems[j])
            return carry

        lax.fori_loop(0, _C // nbuf, step, 0)
        for j in range(nbuf):
            drain_out(orow_bufs[j], osems[j])

    return body(frags, rid, xm, w)


def kernel(images, feature_map):
    f32 = jnp.float32
    fm32 = feature_map.reshape(_NBC, _Y, _X)
    gxr = jnp.linspace(-1.0, 1.0, _X, dtype=f32).reshape(1, _X)
    gyc = jnp.linspace(-1.0, 1.0, _Y, dtype=f32).reshape(_Y, 1)
    g32r = jnp.linspace(-1.0, 1.0, _OX, dtype=f32).reshape(1, _OX)

    mx, my, sc, rid, xm, w = _tc_stats(fm32, gxr, gyc, g32r)
    frags = _tc_retile(images.reshape(_B * _C, _Y, _X))

    out = _sc_bilinear(frags, rid, xm, w)
    mean_x = mx[:, 0, 0].reshape(_B, _FC)
    mean_y = my[:, 0, 0].reshape(_B, _FC)
    scale = sc[:, 0, 0].reshape(_B, _FC)
    return (out, mean_x, mean_y, scale)
